# baseline (speedup 1.0000x reference)
import jax
import jax.numpy as jnp
from jax import lax
from jax.experimental import pallas as pl
from jax.experimental.pallas import tpu as pltpu
from jax.experimental.pallas import tpu_sc as plsc

NC, NS = 2, 16
NW = NC * NS
V = 1000000
D = 32
S0, S1 = 16384, 50
NT_FULL = V // 128
TPW = NT_FULL // NW
JG = 5
TOK = 128
CPW = S0 // (NW * TOK)

_I16 = lambda: lax.iota(jnp.int32, 16)


def _relayout(wt_hbm, tail_hbm, wlin_hbm, bt0, bt1, bo0, bo1,
              ts0, ts1, os0, os1):
    bt = (bt0, bt1)
    bo = (bo0, bo1)
    ts = (ts0, ts1)
    osem = (os0, os1)
    wid = lax.axis_index("s") * NC + lax.axis_index("c")
    base = wid * TPW + 2 * jnp.minimum(wid, 2)
    cnt = TPW + 2 * (wid < 2).astype(jnp.int32)

    def ld(k, b):
        pltpu.async_copy(wt_hbm.at[:, pl.ds((base + k) * 128, 128)],
                         bt[b].at[:, pl.ds(0, 128)], ts[b])

    def wait_ld(b):
        pltpu.make_async_copy(wt_hbm.at[:, pl.ds(0, 128)],
                              bt[b].at[:, pl.ds(0, 128)], ts[b]).wait()

    def st(k, b):
        pltpu.async_copy(bo[b], wlin_hbm.at[pl.ds((base + k) * 32, 32)],
                         osem[b])

    def wait_st(b):
        pltpu.make_async_copy(bo[b], wlin_hbm.at[pl.ds(0, 32)],
                              osem[b]).wait()

    col_idx = (_I16(), _I16() + 16)

    def transpose(b):
        def row(r, vr4):
            for q2 in range(8):
                v = plsc.load_gather(bt[b], [col_idx[q2 % 2],
                                             vr4 + (q2 // 2)])
                bo[b][r, pl.ds(q2 * 16, 16)] = v
            return vr4 + 4
        lax.fori_loop(0, 32, row, jnp.zeros((16,), jnp.int32))

    ld(0, 0)
    ld(1, 1)
    wait_ld(0)
    transpose(0)
    st(0, 0)
    ld(2, 0)
    wait_ld(1)
    transpose(1)
    st(1, 1)
    ld(3, 1)

    def step(k, b):
        wait_ld(b)
        wait_st(b)
        transpose(b)
        st(k, b)
        ld(jnp.minimum(k + 2, cnt - 1), b)

    def pair(p, carry):
        k0 = 2 + 2 * p
        step(k0, 0)
        step(k0 + 1, 1)
        return carry

    lax.fori_loop(0, (cnt - 2) // 2, pair, 0)
    wait_st(0)
    wait_st(1)
    wait_ld(0)
    wait_ld(1)

    @pl.when(wid == NW - 1)
    def _():
        pltpu.sync_copy(tail_hbm, bt0.at[pl.ds(0, 16), pl.ds(0, 128)])
        pltpu.sync_copy(bt0.at[pl.ds(0, 16), pl.ds(0, 128)],
                        wlin_hbm.at[pl.ds(NT_FULL * 32, 16)])


def _gather(w_hbm, g_hbm, out4_hbm,
            idxv, idxT, r0, r1, o0, o1, gs0, gs1, ss0, ss1):
    rows = (r0, r1)
    outst = (o0, o1)
    gs = (gs0, gs1)
    ss = (ss0, ss1)
    wid = lax.axis_index("s") * NC + lax.axis_index("c")

    def fire(jg, b):
        for jj in range(JG):
            pltpu.async_copy(w_hbm.at[idxT.at[jg * JG + jj]],
                             rows[b].at[pl.ds(jj * TOK, TOK)], gs[b])

    def drain(b):
        for jj in range(JG):
            pltpu.make_async_copy(w_hbm.at[pl.ds(0, TOK)],
                                  rows[b].at[pl.ds(jj * TOK, TOK)],
                                  gs[b]).wait()

    def st(jg, ti, b):
        pltpu.async_copy(
            outst[b],
            out4_hbm.at[pl.ds(jg * JG, JG), :, pl.ds(ti, 1), :], ss[b])

    def wait_st(b):
        pltpu.make_async_copy(
            outst[b],
            out4_hbm.at[pl.ds(0, JG), :, pl.ds(0, 1), :], ss[b]).wait()

    lane16 = _I16()

    def transpose(b):
        def drow(d, carry):
            dv = jnp.full((16,), d, jnp.int32)
            td = d // 8
            soff = (d % 8) * 128

            def inner(jj, rbase):
                for k0 in range(0, TOK, 16):
                    v = plsc.load_gather(rows[b], [rbase + k0, dv])
                    outst[b][jj, td, 0, pl.ds(soff + k0, 16)] = v
                return rbase + TOK
            lax.fori_loop(0, JG, inner, lane16)
            return carry
        lax.fori_loop(0, D, drow, 0)

    def chunk(c, carry):
        ti = wid * CPW + c
        pltpu.sync_copy(g_hbm.at[pl.ds(ti * TOK, TOK), :], idxv)

        def trow(j, carry2):
            jv = jnp.full((16,), j, jnp.int32)
            for k0 in range(0, TOK, 16):
                v = plsc.load_gather(idxv, [lane16 + k0, jv])
                idxT[j, pl.ds(k0, 16)] = v
            return carry2
        lax.fori_loop(0, S1, trow, 0)

        NJG = S1 // JG
        for jg in range(NJG):
            b = jg % 2
            if jg >= 2:
                wait_st(b)
            fire(jg, b)
            if jg >= 1:
                drain(1 - b)
                transpose(1 - b)
                st(jg - 1, ti, 1 - b)
        drain(1)
        transpose(1)
        st(NJG - 1, ti, 1)
        wait_st(0)
        wait_st(1)
        return carry

    lax.fori_loop(0, CPW, chunk, 0)


@jax.jit
def kernel(glyph_ids, weight):
    if glyph_ids.dtype != jnp.int32:
        glyph_ids = glyph_ids.astype(jnp.int32)
    mesh = plsc.VectorSubcoreMesh(
        core_axis_name="c", subcore_axis_name="s",
        num_cores=NC, num_subcores=NS,
    )
    wt = weight.T
    tail128 = weight[V - 64:, :].reshape(16, 128)
    wlin = pl.kernel(
        _relayout,
        out_type=jax.ShapeDtypeStruct((V * D // 128, 128), jnp.float32),
        mesh=mesh,
        scratch_types=[
            pltpu.VMEM((32, 129), jnp.float32),
            pltpu.VMEM((32, 129), jnp.float32),
            pltpu.VMEM((32, 128), jnp.float32),
            pltpu.VMEM((32, 128), jnp.float32),
            pltpu.SemaphoreType.DMA,
            pltpu.SemaphoreType.DMA,
            pltpu.SemaphoreType.DMA,
            pltpu.SemaphoreType.DMA,
        ],
        compiler_params=pltpu.CompilerParams(
            use_tc_tiling_on_sc=True, needs_layout_passes=False),
    )(wt, tail128)
    w2 = wlin.reshape(V, D)
    out4 = pl.kernel(
        _gather,
        out_type=jax.ShapeDtypeStruct((S1, D // 8, S0 // 128, 1024),
                                      jnp.float32),
        mesh=mesh,
        scratch_types=[
            pltpu.VMEM((TOK, S1), jnp.int32),
            pltpu.VMEM((S1, TOK), jnp.int32),
            pltpu.VMEM((JG * TOK, D), jnp.float32),
            pltpu.VMEM((JG * TOK, D), jnp.float32),
            pltpu.VMEM((JG, D // 8, 1, 1024), jnp.float32),
            pltpu.VMEM((JG, D // 8, 1, 1024), jnp.float32),
            pltpu.SemaphoreType.DMA,
            pltpu.SemaphoreType.DMA,
            pltpu.SemaphoreType.DMA,
            pltpu.SemaphoreType.DMA,
        ],
        compiler_params=pltpu.CompilerParams(
            use_tc_tiling_on_sc=False, needs_layout_passes=False),
    )(w2, glyph_ids)
    out5 = out4.reshape(S1, D // 8, S0 // 128, 8, 128)
    return out5.transpose(2, 4, 0, 1, 3).reshape(S0, S1, D)


# speedup vs baseline: 1.4199x; 1.4199x over previous
import jax
import jax.numpy as jnp
from jax import lax
from jax.experimental import pallas as pl
from jax.experimental.pallas import tpu as pltpu
from jax.experimental.pallas import tpu_sc as plsc

NC, NS = 2, 16
NW = NC * NS
R = 16


def _body(table_hbm, idx_hbm, out_hbm,
          idx0, idx1, rows0, rows1,
          isem0, isem1, gsem0, gsem1, ssem0, ssem1):
    idx_v = (idx0, idx1)
    rows_v = (rows0, rows1)
    isem = (isem0, isem1)
    gsem = (gsem0, gsem1)
    ssem = (ssem0, ssem1)

    wid = lax.axis_index("s") * NC + lax.axis_index("c")
    n_rows = idx_hbm.shape[0]
    rows_per_w = n_rows // NW
    chunks = rows_per_w // R
    base_row = wid * rows_per_w

    def ld_idx(c, b):
        pltpu.async_copy(idx_hbm.at[pl.ds(base_row + c * R, R)],
                         idx_v[b], isem[b])

    def wait_idx(b):
        pltpu.make_async_copy(idx_hbm.at[pl.ds(0, R)], idx_v[b],
                              isem[b]).wait()

    def fire_gathers(b):
        for rr in range(R):
            pltpu.async_copy(table_hbm.at[idx_v[b].at[rr]],
                             rows_v[b].at[rr], gsem[b])

    def wait_gathers(b):
        pltpu.make_async_copy(out_hbm.at[pl.ds(0, R)], rows_v[b],
                              gsem[b]).wait()

    def st_rows(c, b):
        pltpu.async_copy(rows_v[b], out_hbm.at[pl.ds(base_row + c * R, R)],
                         ssem[b])

    def wait_store(b):
        pltpu.make_async_copy(rows_v[b], out_hbm.at[pl.ds(0, R)],
                              ssem[b]).wait()

    ld_idx(0, 0)
    ld_idx(1, 1)
    wait_idx(0)
    fire_gathers(0)
    wait_idx(1)
    fire_gathers(1)
    wait_gathers(0)
    st_rows(0, 0)
    ld_idx(2, 0)

    def step(c, b):
        wait_idx(b)
        wait_store(b)
        fire_gathers(b)
        wait_gathers(1 - b)
        st_rows(c - 1, 1 - b)
        ld_idx(jnp.minimum(c + 1, chunks - 1), 1 - b)

    def pair(q, carry):
        c0 = 2 + 2 * q
        step(c0, 0)
        step(c0 + 1, 1)
        return carry

    lax.fori_loop(0, (chunks - 2) // 2, pair, 0)

    wait_gathers(1)
    st_rows(chunks - 1, 1)
    wait_idx(0)
    wait_store(0)
    wait_store(1)


@jax.jit
def kernel(glyph_ids, weight):
    S0, S1 = glyph_ids.shape
    D = weight.shape[1]
    if glyph_ids.dtype != jnp.int32:
        glyph_ids = glyph_ids.astype(jnp.int32)
    mesh = plsc.VectorSubcoreMesh(
        core_axis_name="c", subcore_axis_name="s",
        num_cores=NC, num_subcores=NS,
    )
    return pl.kernel(
        _body,
        out_type=jax.ShapeDtypeStruct((S0, S1, D), jnp.float32),
        mesh=mesh,
        scratch_types=[
            pltpu.VMEM((R, S1), jnp.int32),
            pltpu.VMEM((R, S1), jnp.int32),
            pltpu.VMEM((R, S1, D), jnp.float32),
            pltpu.VMEM((R, S1, D), jnp.float32),
            pltpu.SemaphoreType.DMA,
            pltpu.SemaphoreType.DMA,
            pltpu.SemaphoreType.DMA,
            pltpu.SemaphoreType.DMA,
            pltpu.SemaphoreType.DMA,
            pltpu.SemaphoreType.DMA,
        ],
        compiler_params=pltpu.CompilerParams(use_tc_tiling_on_sc=False),
    )(weight, glyph_ids)
